# Initial kernel scaffold; baseline (speedup 1.0000x reference)
import jax
import jax.numpy as jnp
from jax.experimental import pallas as pl


def kernel(user_id, timestamp, user_table, ts_table, buckets):
    raise NotImplementedError("write your pallas kernel here")


# speedup vs baseline: 10.5106x; 10.5106x over previous
import functools

import jax
import jax.numpy as jnp
from jax import lax
from jax.experimental import pallas as pl
from jax.experimental.pallas import tpu as pltpu
from jax.experimental.pallas import tpu_sc as plsc

B = 16384
EMB = 32
NB = 1000
NB_PAD = 1024
VOCAB = 100000
OUT_W = 2 * EMB + 1

NC = 2
NS = 16
L = 16
NW = NC * NS
BPW = B // NW

import numpy as _np

_INV_STD = float(1.0 / _np.sqrt(_np.float32(1.0 / 12.0)))

_mesh = plsc.VectorSubcoreMesh(
    core_axis_name="c", subcore_axis_name="s", num_cores=NC, num_subcores=NS
)


@functools.partial(
    pl.kernel,
    out_type=jax.ShapeDtypeStruct((B, OUT_W), jnp.float32),
    mesh=_mesh,
    compiler_params=pltpu.CompilerParams(
        needs_layout_passes=False, use_tc_tiling_on_sc=False
    ),
    scratch_types=[
        pltpu.VMEM((BPW,), jnp.int32),
        pltpu.VMEM((BPW,), jnp.float32),
        pltpu.VMEM((BPW,), jnp.int32),
        pltpu.VMEM((BPW,), jnp.float32),
        pltpu.VMEM((BPW, EMB), jnp.float32),
        pltpu.VMEM((BPW, EMB), jnp.float32),
        pltpu.VMEM((NB_PAD,), jnp.float32),
        pltpu.VMEM((BPW, OUT_W), jnp.float32),
        pltpu.SemaphoreType.DMA,
        pltpu.SemaphoreType.DMA,
    ],
)
def _user_model_sc(
    uid_hbm, ts_hbm, ut_hbm, tt_hbm, bk_hbm, out_hbm,
    idx_v, ts_v, bidx_v, nrm_v, ue_v, te_v, bk_v, out_v, sem_ue, sem_te,
):
    wid = lax.axis_index("s") * NC + lax.axis_index("c")
    base = wid * BPW

    pltpu.sync_copy(uid_hbm.at[pl.ds(base, BPW)], idx_v)
    ue_cp = pltpu.async_copy(ut_hbm.at[idx_v], ue_v, sem_ue)
    pltpu.sync_copy(ts_hbm.at[pl.ds(base, BPW)], ts_v)
    pltpu.sync_copy(bk_hbm, bk_v)

    iota = lax.iota(jnp.int32, L)

    def search_body(g, carry):
        off = g * L
        t = ts_v[pl.ds(off, L)]
        lo = jnp.zeros((L,), jnp.int32)
        hi = jnp.full((L,), NB, jnp.int32)
        for _ in range(10):
            mid = lax.shift_right_logical(lo + hi, 1)
            bv = plsc.load_gather(bk_v, [mid])
            le = bv <= t
            lo = jnp.where(le, mid + 1, lo)
            hi = jnp.where(le, hi, mid)
        bidx_v[pl.ds(off, L)] = lo
        nrm_v[pl.ds(off, L)] = (t - 0.5) * _INV_STD
        return carry

    lax.fori_loop(0, BPW // L, search_body, 0)

    te_cp = pltpu.async_copy(tt_hbm.at[bidx_v], te_v, sem_te)
    ue_cp.wait()
    te_cp.wait()

    col64 = jnp.full((L,), OUT_W - 1, jnp.int32)

    def asm_body(g, carry):
        off = g * L
        for j in range(L):
            r = off + j
            out_v[r, pl.ds(0, L)] = ue_v[r, pl.ds(0, L)]
            out_v[r, pl.ds(L, L)] = ue_v[r, pl.ds(L, L)]
            out_v[r, pl.ds(2 * L, L)] = te_v[r, pl.ds(0, L)]
            out_v[r, pl.ds(3 * L, L)] = te_v[r, pl.ds(L, L)]
        plsc.store_scatter(out_v, [off + iota, col64], nrm_v[pl.ds(off, L)])
        return carry

    lax.fori_loop(0, BPW // L, asm_body, 0)

    pltpu.sync_copy(out_v, out_hbm.at[pl.ds(base, BPW)])


def kernel(user_id, timestamp, user_table, ts_table, buckets):
    uid = user_id.astype(jnp.int32)
    bk_pad = jnp.concatenate(
        [buckets.astype(jnp.float32), jnp.full((NB_PAD - NB,), 2.0, jnp.float32)]
    )
    return _user_model_sc(uid, timestamp, user_table, ts_table, bk_pad)
